# baseline (speedup 1.0000x reference)
import functools

import jax
import jax.numpy as jnp
from jax import lax
from jax.experimental import pallas as pl
from jax.experimental.pallas import tpu as pltpu
from jax.experimental.pallas import tpu_sc as plsc

N = 10000
M = 128
E = 320000
NUM_LAYERS = 3

NC = 2
NS = 16
B = 128
NB = 80
E_PAD = NC * NS * NB * B
CH = 20
NCH = NB // CH
ROWS_PER_TILE = N // NS
WCH = 25
WOUT = 624

def _sc_layer_body(x_hbm, src_hbm, dst_hbm, adj_hbm, out_hbm,
                   src_v, dst_v, adj_v, rows0_v, zero_v, acc_sh, sem0):
    cid = lax.axis_index("c")
    sid = lax.axis_index("s")

    def _zrow(r, carry):
        for j in range(M // 16):
            zero_v[r, pl.ds(j * 16, 16)] = jnp.zeros((16,), jnp.float32)
        return carry

    lax.fori_loop(0, WCH, _zrow, 0)
    row0 = sid * ROWS_PER_TILE
    for k in range(ROWS_PER_TILE // WCH):
        pltpu.sync_copy(zero_v, acc_sh.at[pl.ds(row0 + k * WCH, WCH)])
    plsc.subcore_barrier()

    def _scale_scatter(rows, b):
        def _scale(g, c2):
            w = adj_v[b, pl.ds(g * 16, 16)]
            for k in range(16):
                s = w[k]
                e = g * 16 + k
                for j in range(M // 16):
                    sl = pl.ds(j * 16, 16)
                    rows[e, sl] = rows[e, sl] * s
            return c2

        lax.fori_loop(0, B // 16, _scale, 0)
        pltpu.sync_copy(rows, acc_sh.at[dst_v.at[b]], add=True)

    def _chunk(ch, carry):
        pltpu.sync_copy(src_hbm.at[cid, sid, ch], src_v)
        pltpu.sync_copy(dst_hbm.at[cid, sid, ch], dst_v)
        pltpu.sync_copy(adj_hbm.at[cid, sid, ch], adj_v)

        def _batch(b, c1):
            pltpu.async_copy(x_hbm.at[src_v.at[b]], rows0_v, sem0).wait()
            _scale_scatter(rows0_v, b)
            return c1

        lax.fori_loop(0, CH, _batch, 0)
        return carry

    lax.fori_loop(0, NCH, _chunk, 0)
    plsc.subcore_barrier()

    w0 = sid * WOUT
    pltpu.sync_copy(acc_sh.at[pl.ds(w0, WOUT)], out_hbm.at[cid, pl.ds(w0, WOUT)])

    @pl.when(sid == NS - 1)
    def _tail():
        t0 = NS * WOUT
        pltpu.sync_copy(acc_sh.at[pl.ds(t0, N - NS * WOUT)],
                        out_hbm.at[cid, pl.ds(t0, N - NS * WOUT)])


@functools.cache
def _sc_layer_kernel():
    mesh = plsc.VectorSubcoreMesh(
        core_axis_name="c", subcore_axis_name="s", num_cores=NC, num_subcores=NS
    )
    return pl.kernel(
        _sc_layer_body,
        out_type=jax.ShapeDtypeStruct((NC, N, M), jnp.float32),
        mesh=mesh,
        scratch_types=[
            pltpu.VMEM((CH, B), jnp.int32),
            pltpu.VMEM((CH, B), jnp.int32),
            pltpu.VMEM((CH, B), jnp.float32),
            pltpu.VMEM((B, M), jnp.float32),
            pltpu.VMEM((WCH, M), jnp.float32),
            pltpu.VMEM_SHARED((N, M), jnp.float32),
            pltpu.SemaphoreType.DMA,
        ],
    )


_R = 1000


def _embs_body(c_ref, a_ref, out_ref):
    c = jnp.tanh(c_ref[...])
    a = a_ref[...]
    r = c[:, 3:4]
    r = jnp.where(a == 2, c[:, 2:3], r)
    r = jnp.where(a == 1, c[:, 1:2], r)
    r = jnp.where(a == 0, c[:, 0:1], r)
    out_ref[...] = r


def _embs_call(centroid_embs, assignment):
    return pl.pallas_call(
        _embs_body,
        grid=(N // _R,),
        in_specs=[
            pl.BlockSpec((_R, 4), lambda i: (i, 0)),
            pl.BlockSpec((_R, M), lambda i: (i, 0)),
        ],
        out_specs=pl.BlockSpec((_R, M), lambda i: (i, 0)),
        out_shape=jax.ShapeDtypeStruct((N, M), jnp.float32),
    )(centroid_embs, assignment)


def _combine_body(scale, p_ref, acc_ref, x_ref, accout_ref):
    x = p_ref[0] + p_ref[1]
    x_ref[...] = x
    accout_ref[...] = (acc_ref[...] + x) * scale


def _combine_call(p, acc, scale):
    return pl.pallas_call(
        functools.partial(_combine_body, scale),
        grid=(N // _R,),
        in_specs=[
            pl.BlockSpec((NC, _R, M), lambda i: (0, i, 0)),
            pl.BlockSpec((_R, M), lambda i: (i, 0)),
        ],
        out_specs=[
            pl.BlockSpec((_R, M), lambda i: (i, 0)),
            pl.BlockSpec((_R, M), lambda i: (i, 0)),
        ],
        out_shape=[
            jax.ShapeDtypeStruct((N, M), jnp.float32),
            jax.ShapeDtypeStruct((N, M), jnp.float32),
        ],
    )(p, acc)


@jax.jit
def kernel(centroid_embs, centroid_assignment, edge_index, adj_values):
    pad = E_PAD - E
    idx_pad = jnp.zeros((2, pad), jnp.int32)
    w_pad = jnp.zeros((pad,), jnp.float32)
    ei = jnp.concatenate([edge_index, idx_pad], axis=1)
    av = jnp.concatenate([adj_values, w_pad])
    src = ei[0].reshape(NC, NS, NCH, CH, B)
    dst = ei[1].reshape(NC, NS, NCH, CH, B)
    adj = av.reshape(NC, NS, NCH, CH, B)

    x = _embs_call(centroid_embs, centroid_assignment)
    acc = x
    for layer in range(NUM_LAYERS):
        p = _sc_layer_kernel()(x, src, dst, adj)
        scale = 0.25 if layer == NUM_LAYERS - 1 else 1.0
        x, acc = _combine_call(p, acc, scale)
    return x, acc


# speedup vs baseline: 2.3420x; 2.3420x over previous
import functools

import jax
import jax.numpy as jnp
from jax import lax
from jax.experimental import pallas as pl
from jax.experimental.pallas import tpu as pltpu
from jax.experimental.pallas import tpu_sc as plsc

N = 10000
M = 128
E = 320000
NUM_LAYERS = 3

NC = 2
NS = 16
B = 128
NB = 80
E_PAD = NC * NS * NB * B
CH = 20
NCH = NB // CH
ROWS_PER_TILE = N // NS
WCH = 25
WOUT = 624

def _sc_layer_body(x_hbm, src_hbm, dst_hbm, adj_hbm, out_hbm,
                   src_v, dst_v, adj_v, rows0_v, zero_v, acc_sh, sem0):
    cid = lax.axis_index("c")
    sid = lax.axis_index("s")

    def _zrow(r, carry):
        for j in range(M // 16):
            zero_v[r, pl.ds(j * 16, 16)] = jnp.zeros((16,), jnp.float32)
        return carry

    lax.fori_loop(0, WCH, _zrow, 0)
    row0 = sid * ROWS_PER_TILE
    for k in range(ROWS_PER_TILE // WCH):
        pltpu.sync_copy(zero_v, acc_sh.at[pl.ds(row0 + k * WCH, WCH)])
    plsc.subcore_barrier()

    def _scale_scatter(rows, b):
        def _scale(g, c2):
            w = adj_v[b, pl.ds(g * 16, 16)]
            for k in range(16):
                s = w[k]
                e = g * 16 + k
                for j in range(M // 16):
                    sl = pl.ds(j * 16, 16)
                    rows[e, sl] = rows[e, sl] * s
            return c2

        lax.fori_loop(0, B // 16, _scale, 0)
        pltpu.sync_copy(rows, acc_sh.at[dst_v.at[b]], add=True)

    def _chunk(ch, carry):
        pltpu.sync_copy(src_hbm.at[cid, sid, ch], src_v)
        pltpu.sync_copy(dst_hbm.at[cid, sid, ch], dst_v)
        pltpu.sync_copy(adj_hbm.at[cid, sid, ch], adj_v)

        def _batch(b, c1):
            pltpu.async_copy(x_hbm.at[src_v.at[b]], rows0_v, sem0).wait()
            _scale_scatter(rows0_v, b)
            return c1

        lax.fori_loop(0, CH, _batch, 0)
        return carry

    lax.fori_loop(0, NCH, _chunk, 0)
    plsc.subcore_barrier()

    w0 = sid * WOUT
    pltpu.sync_copy(acc_sh.at[pl.ds(w0, WOUT)], out_hbm.at[cid, pl.ds(w0, WOUT)])

    @pl.when(sid == NS - 1)
    def _tail():
        t0 = NS * WOUT
        pltpu.sync_copy(acc_sh.at[pl.ds(t0, N - NS * WOUT)],
                        out_hbm.at[cid, pl.ds(t0, N - NS * WOUT)])


@functools.cache
def _sc_layer_kernel():
    mesh = plsc.VectorSubcoreMesh(
        core_axis_name="c", subcore_axis_name="s", num_cores=NC, num_subcores=NS
    )
    return pl.kernel(
        _sc_layer_body,
        out_type=jax.ShapeDtypeStruct((NC, N, M), jnp.float32),
        mesh=mesh,
        scratch_types=[
            pltpu.VMEM((CH, B), jnp.int32),
            pltpu.VMEM((CH, B), jnp.int32),
            pltpu.VMEM((CH, B), jnp.float32),
            pltpu.VMEM((B, M), jnp.float32),
            pltpu.VMEM((WCH, M), jnp.float32),
            pltpu.VMEM_SHARED((N, M), jnp.float32),
            pltpu.SemaphoreType.DMA,
        ],
    )


_R = 1000


def _embs_body(c_ref, a_ref, out_ref):
    c = jnp.tanh(c_ref[...])
    a = a_ref[...]
    r = c[:, 3:4]
    r = jnp.where(a == 2, c[:, 2:3], r)
    r = jnp.where(a == 1, c[:, 1:2], r)
    r = jnp.where(a == 0, c[:, 0:1], r)
    out_ref[...] = r


def _embs_call(centroid_embs, assignment):
    return pl.pallas_call(
        _embs_body,
        grid=(N // _R,),
        in_specs=[
            pl.BlockSpec((_R, 4), lambda i: (i, 0)),
            pl.BlockSpec((_R, M), lambda i: (i, 0)),
        ],
        out_specs=pl.BlockSpec((_R, M), lambda i: (i, 0)),
        out_shape=jax.ShapeDtypeStruct((N, M), jnp.float32),
    )(centroid_embs, assignment)


def _combine_body(scale, p_ref, acc_ref, x_ref, accout_ref):
    x = p_ref[0] + p_ref[1]
    x_ref[...] = x
    accout_ref[...] = (acc_ref[...] + x) * scale


def _combine_call(p, acc, scale):
    return pl.pallas_call(
        functools.partial(_combine_body, scale),
        grid=(N // _R,),
        in_specs=[
            pl.BlockSpec((NC, _R, M), lambda i: (0, i, 0)),
            pl.BlockSpec((_R, M), lambda i: (i, 0)),
        ],
        out_specs=[
            pl.BlockSpec((_R, M), lambda i: (i, 0)),
            pl.BlockSpec((_R, M), lambda i: (i, 0)),
        ],
        out_shape=[
            jax.ShapeDtypeStruct((N, M), jnp.float32),
            jax.ShapeDtypeStruct((N, M), jnp.float32),
        ],
    )(p, acc)


@jax.jit
def kernel(centroid_embs, centroid_assignment, edge_index, adj_values):
    pad = E_PAD - E
    spread = jnp.arange(pad, dtype=jnp.int32) % N
    idx_pad = jnp.stack([spread, spread])
    w_pad = jnp.zeros((pad,), jnp.float32)
    ei = jnp.concatenate([edge_index, idx_pad], axis=1)
    av = jnp.concatenate([adj_values, w_pad])
    src = ei[0].reshape(NC, NS, NCH, CH, B)
    dst = ei[1].reshape(NC, NS, NCH, CH, B)
    adj = av.reshape(NC, NS, NCH, CH, B)

    x = _embs_call(centroid_embs, centroid_assignment)
    acc = x
    for layer in range(NUM_LAYERS):
        p = _sc_layer_kernel()(x, src, dst, adj)
        scale = 0.25 if layer == NUM_LAYERS - 1 else 1.0
        x, acc = _combine_call(p, acc, scale)
    return x, acc


# speedup vs baseline: 2.6196x; 1.1185x over previous
import functools

import jax
import jax.numpy as jnp
from jax import lax
from jax.experimental import pallas as pl
from jax.experimental.pallas import tpu as pltpu
from jax.experimental.pallas import tpu_sc as plsc

N = 10000
M = 128
E = 320000
NUM_LAYERS = 3

NC = 2
NS = 16
B = 128
NB = 80
E_PAD = NC * NS * NB * B
CH = 20
NCH = NB // CH
ROWS_PER_TILE = N // NS
WCH = 25
WOUT = 624

def _sc_layer_body(x_hbm, src_hbm, dst_hbm, adj_hbm, out_hbm,
                   src_v, dst_v, adj_v, rows0_v, rows1_v, zero_v, acc_sh,
                   sem0, sem1):
    cid = lax.axis_index("c")
    sid = lax.axis_index("s")

    def _zrow(r, carry):
        for j in range(M // 16):
            zero_v[r, pl.ds(j * 16, 16)] = jnp.zeros((16,), jnp.float32)
        return carry

    lax.fori_loop(0, WCH, _zrow, 0)
    row0 = sid * ROWS_PER_TILE
    for k in range(ROWS_PER_TILE // WCH):
        pltpu.sync_copy(zero_v, acc_sh.at[pl.ds(row0 + k * WCH, WCH)])
    plsc.subcore_barrier()

    def _scale_scatter(rows, b):
        def _scale(g, c2):
            w = adj_v[b, pl.ds(g * 16, 16)]
            for k in range(16):
                s = w[k]
                e = g * 16 + k
                for j in range(M // 16):
                    sl = pl.ds(j * 16, 16)
                    rows[e, sl] = rows[e, sl] * s
            return c2

        lax.fori_loop(0, B // 16, _scale, 0)
        pltpu.sync_copy(rows, acc_sh.at[dst_v.at[b]], add=True)

    def _chunk(ch, carry):
        pltpu.sync_copy(src_hbm.at[cid, sid, ch], src_v)
        pltpu.sync_copy(dst_hbm.at[cid, sid, ch], dst_v)
        pltpu.sync_copy(adj_hbm.at[cid, sid, ch], adj_v)

        def _pair(i, c1):
            b = 2 * i
            cp0 = pltpu.async_copy(x_hbm.at[src_v.at[b]], rows0_v, sem0)
            cp1 = pltpu.async_copy(x_hbm.at[src_v.at[b + 1]], rows1_v, sem1)
            cp0.wait()
            _scale_scatter(rows0_v, b)
            cp1.wait()
            _scale_scatter(rows1_v, b + 1)
            return c1

        lax.fori_loop(0, CH // 2, _pair, 0)
        return carry

    lax.fori_loop(0, NCH, _chunk, 0)
    plsc.subcore_barrier()

    w0 = sid * WOUT
    pltpu.sync_copy(acc_sh.at[pl.ds(w0, WOUT)], out_hbm.at[cid, pl.ds(w0, WOUT)])

    @pl.when(sid == NS - 1)
    def _tail():
        t0 = NS * WOUT
        pltpu.sync_copy(acc_sh.at[pl.ds(t0, N - NS * WOUT)],
                        out_hbm.at[cid, pl.ds(t0, N - NS * WOUT)])


@functools.cache
def _sc_layer_kernel():
    mesh = plsc.VectorSubcoreMesh(
        core_axis_name="c", subcore_axis_name="s", num_cores=NC, num_subcores=NS
    )
    return pl.kernel(
        _sc_layer_body,
        out_type=jax.ShapeDtypeStruct((NC, N, M), jnp.float32),
        mesh=mesh,
        scratch_types=[
            pltpu.VMEM((CH, B), jnp.int32),
            pltpu.VMEM((CH, B), jnp.int32),
            pltpu.VMEM((CH, B), jnp.float32),
            pltpu.VMEM((B, M), jnp.float32),
            pltpu.VMEM((B, M), jnp.float32),
            pltpu.VMEM((WCH, M), jnp.float32),
            pltpu.VMEM_SHARED((N, M), jnp.float32),
            pltpu.SemaphoreType.DMA,
            pltpu.SemaphoreType.DMA,
        ],
    )


_R = 1000


def _embs_body(c_ref, a_ref, out_ref):
    c = jnp.tanh(c_ref[...])
    a = a_ref[...]
    r = c[:, 3:4]
    r = jnp.where(a == 2, c[:, 2:3], r)
    r = jnp.where(a == 1, c[:, 1:2], r)
    r = jnp.where(a == 0, c[:, 0:1], r)
    out_ref[...] = r


def _embs_call(centroid_embs, assignment):
    return pl.pallas_call(
        _embs_body,
        grid=(N // _R,),
        in_specs=[
            pl.BlockSpec((_R, 4), lambda i: (i, 0)),
            pl.BlockSpec((_R, M), lambda i: (i, 0)),
        ],
        out_specs=pl.BlockSpec((_R, M), lambda i: (i, 0)),
        out_shape=jax.ShapeDtypeStruct((N, M), jnp.float32),
    )(centroid_embs, assignment)


def _combine_body(scale, p_ref, acc_ref, x_ref, accout_ref):
    x = p_ref[0] + p_ref[1]
    x_ref[...] = x
    accout_ref[...] = (acc_ref[...] + x) * scale


def _combine_call(p, acc, scale):
    return pl.pallas_call(
        functools.partial(_combine_body, scale),
        grid=(N // _R,),
        in_specs=[
            pl.BlockSpec((NC, _R, M), lambda i: (0, i, 0)),
            pl.BlockSpec((_R, M), lambda i: (i, 0)),
        ],
        out_specs=[
            pl.BlockSpec((_R, M), lambda i: (i, 0)),
            pl.BlockSpec((_R, M), lambda i: (i, 0)),
        ],
        out_shape=[
            jax.ShapeDtypeStruct((N, M), jnp.float32),
            jax.ShapeDtypeStruct((N, M), jnp.float32),
        ],
    )(p, acc)


@jax.jit
def kernel(centroid_embs, centroid_assignment, edge_index, adj_values):
    pad = E_PAD - E
    spread = jnp.arange(pad, dtype=jnp.int32) % N
    idx_pad = jnp.stack([spread, spread])
    w_pad = jnp.zeros((pad,), jnp.float32)
    ei = jnp.concatenate([edge_index, idx_pad], axis=1)
    av = jnp.concatenate([adj_values, w_pad])
    src = ei[0].reshape(NC, NS, NCH, CH, B)
    dst = ei[1].reshape(NC, NS, NCH, CH, B)
    adj = av.reshape(NC, NS, NCH, CH, B)

    x = _embs_call(centroid_embs, centroid_assignment)
    acc = x
    for layer in range(NUM_LAYERS):
        p = _sc_layer_kernel()(x, src, dst, adj)
        scale = 0.25 if layer == NUM_LAYERS - 1 else 1.0
        x, acc = _combine_call(p, acc, scale)
    return x, acc


# speedup vs baseline: 3.4003x; 1.2980x over previous
import functools

import jax
import jax.numpy as jnp
from jax import lax
from jax.experimental import pallas as pl
from jax.experimental.pallas import tpu as pltpu
from jax.experimental.pallas import tpu_sc as plsc

N = 10000
M = 128
E = 320000
NUM_LAYERS = 3

NC = 2
NS = 16
B = 128
NB = 80
E_PAD = NC * NS * NB * B
CH = 20
NCH = NB // CH
ROWS_PER_TILE = N // NS
WCH = 25
WOUT = 624

def _sc_layer_body(x_hbm, src_hbm, dst_hbm, adj_hbm, out_hbm,
                   src_v, dst_v, adj_v, rows0_v, rows1_v, zero_v, acc_sh,
                   sem0, sem1):
    cid = lax.axis_index("c")
    sid = lax.axis_index("s")

    def _zrow(r, carry):
        for j in range(M // 16):
            zero_v[r, pl.ds(j * 16, 16)] = jnp.zeros((16,), jnp.float32)
        return carry

    lax.fori_loop(0, WCH, _zrow, 0)
    row0 = sid * ROWS_PER_TILE
    for k in range(ROWS_PER_TILE // WCH):
        pltpu.sync_copy(zero_v, acc_sh.at[pl.ds(row0 + k * WCH, WCH)])
    plsc.subcore_barrier()

    def _scale_scatter(rows, b):
        def _scale(g, c2):
            w = adj_v[b, pl.ds(g * 16, 16)]
            for k in range(16):
                s = w[k]
                e = g * 16 + k
                for j in range(M // 16):
                    sl = pl.ds(j * 16, 16)
                    rows[e, sl] = rows[e, sl] * s
            return c2

        lax.fori_loop(0, B // 16, _scale, 0)
        pltpu.sync_copy(rows, acc_sh.at[dst_v.at[b]], add=True)

    def _chunk(ch, carry):
        pltpu.sync_copy(src_hbm.at[cid, sid, ch], src_v)
        pltpu.sync_copy(dst_hbm.at[cid, sid, ch], dst_v)
        pltpu.sync_copy(adj_hbm.at[cid, sid, ch], adj_v)

        bufs = (rows0_v, rows1_v)
        sems = (sem0, sem1)
        cps = [pltpu.async_copy(x_hbm.at[src_v.at[0]], rows0_v, sem0),
               pltpu.async_copy(x_hbm.at[src_v.at[1]], rows1_v, sem1)]
        for b in range(CH):
            p = b % 2
            cps[p].wait()
            _scale_scatter(bufs[p], b)
            if b + 2 < CH:
                cps[p] = pltpu.async_copy(
                    x_hbm.at[src_v.at[b + 2]], bufs[p], sems[p])
        return carry

    lax.fori_loop(0, NCH, _chunk, 0)
    plsc.subcore_barrier()

    w0 = sid * WOUT
    pltpu.sync_copy(acc_sh.at[pl.ds(w0, WOUT)], out_hbm.at[cid, pl.ds(w0, WOUT)])

    @pl.when(sid == NS - 1)
    def _tail():
        t0 = NS * WOUT
        pltpu.sync_copy(acc_sh.at[pl.ds(t0, N - NS * WOUT)],
                        out_hbm.at[cid, pl.ds(t0, N - NS * WOUT)])


@functools.cache
def _sc_layer_kernel():
    mesh = plsc.VectorSubcoreMesh(
        core_axis_name="c", subcore_axis_name="s", num_cores=NC, num_subcores=NS
    )
    return pl.kernel(
        _sc_layer_body,
        out_type=jax.ShapeDtypeStruct((NC, N, M), jnp.float32),
        mesh=mesh,
        scratch_types=[
            pltpu.VMEM((CH, B), jnp.int32),
            pltpu.VMEM((CH, B), jnp.int32),
            pltpu.VMEM((CH, B), jnp.float32),
            pltpu.VMEM((B, M), jnp.float32),
            pltpu.VMEM((B, M), jnp.float32),
            pltpu.VMEM((WCH, M), jnp.float32),
            pltpu.VMEM_SHARED((N, M), jnp.float32),
            pltpu.SemaphoreType.DMA,
            pltpu.SemaphoreType.DMA,
        ],
    )


_R = 1000


def _embs_body(c_ref, a_ref, out_ref):
    c = jnp.tanh(c_ref[...])
    a = a_ref[...]
    r = c[:, 3:4]
    r = jnp.where(a == 2, c[:, 2:3], r)
    r = jnp.where(a == 1, c[:, 1:2], r)
    r = jnp.where(a == 0, c[:, 0:1], r)
    out_ref[...] = r


def _embs_call(centroid_embs, assignment):
    return pl.pallas_call(
        _embs_body,
        grid=(N // _R,),
        in_specs=[
            pl.BlockSpec((_R, 4), lambda i: (i, 0)),
            pl.BlockSpec((_R, M), lambda i: (i, 0)),
        ],
        out_specs=pl.BlockSpec((_R, M), lambda i: (i, 0)),
        out_shape=jax.ShapeDtypeStruct((N, M), jnp.float32),
    )(centroid_embs, assignment)


def _combine_body(scale, p_ref, acc_ref, x_ref, accout_ref):
    x = p_ref[0] + p_ref[1]
    x_ref[...] = x
    accout_ref[...] = (acc_ref[...] + x) * scale


def _combine_call(p, acc, scale):
    return pl.pallas_call(
        functools.partial(_combine_body, scale),
        grid=(N // _R,),
        in_specs=[
            pl.BlockSpec((NC, _R, M), lambda i: (0, i, 0)),
            pl.BlockSpec((_R, M), lambda i: (i, 0)),
        ],
        out_specs=[
            pl.BlockSpec((_R, M), lambda i: (i, 0)),
            pl.BlockSpec((_R, M), lambda i: (i, 0)),
        ],
        out_shape=[
            jax.ShapeDtypeStruct((N, M), jnp.float32),
            jax.ShapeDtypeStruct((N, M), jnp.float32),
        ],
    )(p, acc)


@jax.jit
def kernel(centroid_embs, centroid_assignment, edge_index, adj_values):
    pad = E_PAD - E
    spread = jnp.arange(pad, dtype=jnp.int32) % N
    idx_pad = jnp.stack([spread, spread])
    w_pad = jnp.zeros((pad,), jnp.float32)
    ei = jnp.concatenate([edge_index, idx_pad], axis=1)
    av = jnp.concatenate([adj_values, w_pad])
    src = ei[0].reshape(NC, NS, NCH, CH, B)
    dst = ei[1].reshape(NC, NS, NCH, CH, B)
    adj = av.reshape(NC, NS, NCH, CH, B)

    x = _embs_call(centroid_embs, centroid_assignment)
    acc = x
    for layer in range(NUM_LAYERS):
        p = _sc_layer_kernel()(x, src, dst, adj)
        scale = 0.25 if layer == NUM_LAYERS - 1 else 1.0
        x, acc = _combine_call(p, acc, scale)
    return x, acc
